# Initial kernel scaffold; baseline (speedup 1.0000x reference)
import jax
import jax.numpy as jnp
from jax.experimental import pallas as pl


def kernel(x, edge_index, w1, b1, g1, be1, rm1, rv1, w2, b2, g2, be2, rm2, rv2):
    raise NotImplementedError("write your pallas kernel here")


# speedup vs baseline: 4.3738x; 4.3738x over previous
import functools

import jax
import jax.numpy as jnp
from jax import lax
from jax.experimental import pallas as pl
from jax.experimental.pallas import tpu as pltpu
from jax.experimental.pallas import tpu_sc as plsc

N = 10000
E = 320000
D = 128
LAYERS = 4
BN_EPS = 1e-5

NC = 2
NS = 16
K = 80
PER_TILE = E // (NC * NS)
CHUNKS = PER_TILE // K
ROWS_PER_TILE = 624
ROWS_TAIL = N - NS * ROWS_PER_TILE

_mesh = plsc.VectorSubcoreMesh(core_axis_name="c", subcore_axis_name="s")


@functools.partial(
    pl.kernel,
    mesh=_mesh,
    out_type=jax.ShapeDtypeStruct((NC, N, D), jnp.float32),
    scratch_types=[
        pltpu.VMEM((K,), jnp.int32),
        pltpu.VMEM((K,), jnp.int32),
        pltpu.VMEM((K, D), jnp.float32),
        pltpu.VMEM_SHARED((N, D), jnp.float32),
        pltpu.SemaphoreType.DMA,
    ],
)
def _sc_agg(h_hbm, src_hbm, dst_hbm, zeros_hbm, out_hbm,
            src_v, dst_v, rows_v, agg_sh, sem):
    c = lax.axis_index("c")
    s = lax.axis_index("s")

    @pl.when(s == 0)
    def _zero():
        pltpu.sync_copy(zeros_hbm, agg_sh)

    plsc.subcore_barrier()

    base0 = (c * NS + s) * PER_TILE

    def body(i, carry):
        b = pl.multiple_of(base0 + i * K, 8)
        pltpu.sync_copy(src_hbm.at[pl.ds(b, K)], src_v)
        pltpu.sync_copy(dst_hbm.at[pl.ds(b, K)], dst_v)
        pltpu.async_copy(h_hbm.at[src_v], rows_v, sem).wait()
        pltpu.sync_copy(rows_v, agg_sh.at[dst_v], add=True)
        return carry

    lax.fori_loop(0, CHUNKS, body, 0)

    plsc.subcore_barrier()
    r0 = s * ROWS_PER_TILE
    pltpu.sync_copy(agg_sh.at[pl.ds(r0, ROWS_PER_TILE)],
                    out_hbm.at[c, pl.ds(r0, ROWS_PER_TILE)])

    @pl.when(s == NS - 1)
    def _tail():
        rt = NS * ROWS_PER_TILE
        pltpu.sync_copy(agg_sh.at[pl.ds(rt, ROWS_TAIL)],
                        out_hbm.at[c, pl.ds(rt, ROWS_TAIL)])


BLK = 1000


def _mlp_body(h_ref, a0_ref, a1_ref, w1_ref, s1_ref, t1_ref,
              w2_ref, s2_ref, t2_ref, o_ref, *, final_relu):
    z = h_ref[...] + a0_ref[...] + a1_ref[...]
    z = jnp.dot(z, w1_ref[...], preferred_element_type=jnp.float32)
    z = z * s1_ref[...] + t1_ref[...]
    z = jnp.maximum(z, 0.0)
    z = jnp.dot(z, w2_ref[...], preferred_element_type=jnp.float32)
    z = z * s2_ref[...] + t2_ref[...]
    if final_relu:
        z = jnp.maximum(z, 0.0)
    o_ref[...] = z


def _mlp(h, a0, a1, w1, s1, t1, w2, s2, t2, final_relu):
    row = lambda i: (i, 0)
    fixed = lambda i: (0, 0)
    return pl.pallas_call(
        functools.partial(_mlp_body, final_relu=final_relu),
        grid=(N // BLK,),
        in_specs=[
            pl.BlockSpec((BLK, D), row),
            pl.BlockSpec((BLK, D), row),
            pl.BlockSpec((BLK, D), row),
            pl.BlockSpec((D, D), fixed),
            pl.BlockSpec((1, D), fixed),
            pl.BlockSpec((1, D), fixed),
            pl.BlockSpec((D, D), fixed),
            pl.BlockSpec((1, D), fixed),
            pl.BlockSpec((1, D), fixed),
        ],
        out_specs=pl.BlockSpec((BLK, D), row),
        out_shape=jax.ShapeDtypeStruct((N, D), jnp.float32),
    )(h, a0, a1, w1, s1, t1, w2, s2, t2)


def kernel(x, edge_index, w1, b1, g1, be1, rm1, rv1, w2, b2, g2, be2, rm2, rv2):
    src = edge_index[0].astype(jnp.int32)
    dst = edge_index[1].astype(jnp.int32)
    s1 = g1 * lax.rsqrt(rv1 + BN_EPS)
    t1 = (b1 - rm1) * s1 + be1
    s2 = g2 * lax.rsqrt(rv2 + BN_EPS)
    t2 = (b2 - rm2) * s2 + be2
    zeros = jnp.zeros((N, D), jnp.float32)
    h = x.astype(jnp.float32)
    for l in range(LAYERS):
        parts = _sc_agg(h, src, dst, zeros)
        h = _mlp(h, parts[0], parts[1], w1[l],
                 s1[l][None, :], t1[l][None, :],
                 w2[l], s2[l][None, :], t2[l][None, :],
                 l < LAYERS - 1)
    return h


# speedup vs baseline: 8.3159x; 1.9013x over previous
import functools

import jax
import jax.numpy as jnp
from jax import lax
from jax.experimental import pallas as pl
from jax.experimental.pallas import tpu as pltpu
from jax.experimental.pallas import tpu_sc as plsc

N = 10000
E = 320000
D = 128
LAYERS = 4
BN_EPS = 1e-5

NC = 2
NS = 16
NT = NC * NS
K = 80
PER_TILE = E // NT
CHUNKS = PER_TILE // K
NBUF = 3
PEEL = CHUNKS % NBUF
GROUPS = (CHUNKS - PEEL) // NBUF
ROWS_PER_TILE = 624
ROWS_TAIL = N - NS * ROWS_PER_TILE

_mesh = plsc.VectorSubcoreMesh(core_axis_name="c", subcore_axis_name="s")


@functools.partial(
    pl.kernel,
    mesh=_mesh,
    out_type=jax.ShapeDtypeStruct((NC, N, D), jnp.float32),
    scratch_types=[
        pltpu.VMEM((NBUF, 2, K), jnp.int32),
        pltpu.VMEM((NBUF, K, D), jnp.float32),
        pltpu.VMEM_SHARED((N, D), jnp.float32),
    ]
    + [pltpu.SemaphoreType.DMA] * (3 * NBUF),
)
def _sc_agg(h_hbm, idx_hbm, zeros_hbm, out_hbm,
            idx_v, rows_v, agg_sh, *sems):
    c = lax.axis_index("c")
    s = lax.axis_index("s")
    sem_i = sems[:NBUF]
    sem_g = sems[NBUF:2 * NBUF]
    sem_s = sems[2 * NBUF:]
    tid = c * NS + s

    r0 = s * ROWS_PER_TILE
    pltpu.sync_copy(zeros_hbm.at[pl.ds(0, ROWS_PER_TILE)],
                    agg_sh.at[pl.ds(r0, ROWS_PER_TILE)])

    @pl.when(s == NS - 1)
    def _zero_tail():
        rt = NS * ROWS_PER_TILE
        pltpu.sync_copy(zeros_hbm.at[pl.ds(0, ROWS_TAIL)],
                        agg_sh.at[pl.ds(rt, ROWS_TAIL)])

    plsc.subcore_barrier()

    def _idx(chunk, b):
        return pltpu.make_async_copy(idx_hbm.at[tid, chunk], idx_v.at[b],
                                     sem_i[b])

    def _gather(b):
        return pltpu.make_async_copy(h_hbm.at[idx_v.at[b, 0]],
                                     rows_v.at[b], sem_g[b])

    def _scatter(b):
        return pltpu.make_async_copy(rows_v.at[b],
                                     agg_sh.at[idx_v.at[b, 1]], sem_s[b])

    for ch in range(PEEL):
        _idx(ch, 0).start()
        _idx(ch, 0).wait()
        _gather(0).start()
        _gather(0).wait()
        _scatter(0).start(add=True)
        _scatter(0).wait()
    for b in range(NBUF):
        _idx(PEEL + b, b).start()
        _idx(PEEL + b, b).wait()
        _gather(b).start()

    def body(j, carry):
        for b in range(NBUF):
            _gather(b).wait()
            _scatter(b).start(add=True)
        for b in range(NBUF):
            ch = PEEL + j * NBUF + b
            _scatter(b).wait()
            _idx(ch + NBUF, b).start()
        for b in range(NBUF):
            _idx(PEEL, b).wait()
            _gather(b).start()
        return carry

    lax.fori_loop(0, GROUPS - 1, body, 0)

    for b in range(NBUF):
        _gather(b).wait()
        _scatter(b).start(add=True)
    for b in range(NBUF):
        _scatter(b).wait()

    plsc.subcore_barrier()
    pltpu.sync_copy(agg_sh.at[pl.ds(r0, ROWS_PER_TILE)],
                    out_hbm.at[c, pl.ds(r0, ROWS_PER_TILE)])

    @pl.when(s == NS - 1)
    def _tail():
        rt = NS * ROWS_PER_TILE
        pltpu.sync_copy(agg_sh.at[pl.ds(rt, ROWS_TAIL)],
                        out_hbm.at[c, pl.ds(rt, ROWS_TAIL)])


BLK = 1000


def _mlp_body(h_ref, a0_ref, a1_ref, w1_ref, s1_ref, t1_ref,
              w2_ref, s2_ref, t2_ref, o_ref, *, final_relu):
    z = h_ref[...] + a0_ref[...] + a1_ref[...]
    z = jnp.dot(z, w1_ref[...], preferred_element_type=jnp.float32)
    z = z * s1_ref[...] + t1_ref[...]
    z = jnp.maximum(z, 0.0)
    z = jnp.dot(z, w2_ref[...], preferred_element_type=jnp.float32)
    z = z * s2_ref[...] + t2_ref[...]
    if final_relu:
        z = jnp.maximum(z, 0.0)
    o_ref[...] = z


def _mlp(h, a0, a1, w1, s1, t1, w2, s2, t2, final_relu):
    row = lambda i: (i, 0)
    fixed = lambda i: (0, 0)
    return pl.pallas_call(
        functools.partial(_mlp_body, final_relu=final_relu),
        grid=(N // BLK,),
        in_specs=[
            pl.BlockSpec((BLK, D), row),
            pl.BlockSpec((BLK, D), row),
            pl.BlockSpec((BLK, D), row),
            pl.BlockSpec((D, D), fixed),
            pl.BlockSpec((1, D), fixed),
            pl.BlockSpec((1, D), fixed),
            pl.BlockSpec((D, D), fixed),
            pl.BlockSpec((1, D), fixed),
            pl.BlockSpec((1, D), fixed),
        ],
        out_specs=pl.BlockSpec((BLK, D), row),
        out_shape=jax.ShapeDtypeStruct((N, D), jnp.float32),
    )(h, a0, a1, w1, s1, t1, w2, s2, t2)


def kernel(x, edge_index, w1, b1, g1, be1, rm1, rv1, w2, b2, g2, be2, rm2, rv2):
    src = edge_index[0].astype(jnp.int32).reshape(NT, CHUNKS, K)
    dst = edge_index[1].astype(jnp.int32).reshape(NT, CHUNKS, K)
    idx = jnp.stack([src, dst], axis=2)
    s1 = g1 * lax.rsqrt(rv1 + BN_EPS)
    t1 = (b1 - rm1) * s1 + be1
    s2 = g2 * lax.rsqrt(rv2 + BN_EPS)
    t2 = (b2 - rm2) * s2 + be2
    zeros = jnp.zeros((ROWS_PER_TILE, D), jnp.float32)
    h = x.astype(jnp.float32)
    for l in range(LAYERS):
        parts = _sc_agg(h, idx, zeros)
        h = _mlp(h, parts[0], parts[1], w1[l],
                 s1[l][None, :], t1[l][None, :],
                 w2[l], s2[l][None, :], t2[l][None, :],
                 l < LAYERS - 1)
    return h
